# baseline (speedup 1.0000x reference)
import functools

import jax
import jax.numpy as jnp
from jax import lax
from jax.experimental import pallas as pl
from jax.experimental.pallas import tpu as pltpu
from jax.experimental.pallas import tpu_sc as plsc

D_MODEL = 64
MAX_LEN = 200
BATCH = 4096
SEQ = 200


@functools.lru_cache(maxsize=None)
def _make_sc_gather(V, D, B):
    NC, NS = 2, 16
    NW = NC * NS
    assert B % (8 * NW) == 0
    b_per_w = B // NW
    mesh = plsc.VectorSubcoreMesh(
        core_axis_name="c", subcore_axis_name="s", num_cores=NC, num_subcores=NS
    )

    @functools.partial(
        pl.kernel,
        mesh=mesh,
        out_type=jax.ShapeDtypeStruct((B, D), jnp.float32),
        scratch_types=[
            pltpu.VMEM((b_per_w,), jnp.int32),
            pltpu.VMEM((b_per_w, D), jnp.float32),
            pltpu.SemaphoreType.DMA,
        ],
    )
    def gather(table_hbm, idx_hbm, out_hbm, idx_v, rows_v, sem):
        wid = lax.axis_index("s") * NC + lax.axis_index("c")
        base = wid * b_per_w
        pltpu.sync_copy(idx_hbm.at[pl.ds(base, b_per_w)], idx_v)
        pltpu.async_copy(table_hbm.at[idx_v], rows_v, sem).wait()
        pltpu.sync_copy(rows_v, out_hbm.at[pl.ds(base, b_per_w)])

    return gather


def _add_body(x_ref, pos_ref, o_ref):
    o_ref[...] = x_ref[...] + pos_ref[:, :D_MODEL][:, None, :]


def _tc_add(x, pos, rows_per_block=32):
    B, S, D = x.shape
    grid = (B // rows_per_block,)
    return pl.pallas_call(
        _add_body,
        grid=grid,
        in_specs=[
            pl.BlockSpec((rows_per_block, S, D), lambda i: (i, 0, 0)),
            pl.BlockSpec((rows_per_block, pos.shape[1]), lambda i: (i, 0)),
        ],
        out_specs=pl.BlockSpec((rows_per_block, S, D), lambda i: (i, 0, 0)),
        out_shape=jax.ShapeDtypeStruct((B, S, D), x.dtype),
    )(x, pos)


@jax.jit
def kernel(x, pe, step_indices):
    idx = jnp.clip(step_indices.reshape(-1).astype(jnp.int32), 0, MAX_LEN - 1)
    table = jnp.pad(pe[0], ((0, 0), (0, 128 - D_MODEL)))
    pos = _make_sc_gather(MAX_LEN, 128, BATCH)(table, idx)
    return _tc_add(x, pos)


# speedup vs baseline: 1.5074x; 1.5074x over previous
import functools

import jax
import jax.numpy as jnp
from jax import lax
from jax.experimental import pallas as pl
from jax.experimental.pallas import tpu as pltpu
from jax.experimental.pallas import tpu_sc as plsc

D_MODEL = 64
MAX_LEN = 200
BATCH = 4096
SEQ = 200


@functools.lru_cache(maxsize=None)
def _make_sc_gather(V, D, B):
    NC, NS = 2, 16
    NW = NC * NS
    assert B % (8 * NW) == 0
    b_per_w = B // NW
    mesh = plsc.VectorSubcoreMesh(
        core_axis_name="c", subcore_axis_name="s", num_cores=NC, num_subcores=NS
    )

    @functools.partial(
        pl.kernel,
        mesh=mesh,
        out_type=jax.ShapeDtypeStruct((B, D), jnp.float32),
        scratch_types=[
            pltpu.VMEM((b_per_w,), jnp.int32),
            pltpu.VMEM((b_per_w, D), jnp.float32),
            pltpu.SemaphoreType.DMA,
        ],
    )
    def gather(table_hbm, idx_hbm, out_hbm, idx_v, rows_v, sem):
        wid = lax.axis_index("s") * NC + lax.axis_index("c")
        base = wid * b_per_w
        pltpu.sync_copy(idx_hbm.at[pl.ds(base, b_per_w)], idx_v)
        pltpu.async_copy(table_hbm.at[idx_v], rows_v, sem).wait()
        pltpu.sync_copy(rows_v, out_hbm.at[pl.ds(base, b_per_w)])

    return gather


def _add_body(x_ref, pos_ref, o_ref):
    o_ref[...] = x_ref[...] + pos_ref[...][:, None, :]


def _tc_add(x, pos, rows_per_block=32):
    B, S, D = x.shape
    grid = (B // rows_per_block,)
    return pl.pallas_call(
        _add_body,
        grid=grid,
        in_specs=[
            pl.BlockSpec((rows_per_block, S, D), lambda i: (i, 0, 0)),
            pl.BlockSpec((rows_per_block, pos.shape[1]), lambda i: (i, 0)),
        ],
        out_specs=pl.BlockSpec((rows_per_block, S, D), lambda i: (i, 0, 0)),
        out_shape=jax.ShapeDtypeStruct((B, S, D), x.dtype),
    )(x, pos)


@jax.jit
def kernel(x, pe, step_indices):
    idx = jnp.clip(step_indices.reshape(-1).astype(jnp.int32), 0, MAX_LEN - 1)
    table = jnp.concatenate([pe[0], pe[0]], axis=1)
    pos = _make_sc_gather(MAX_LEN, 128, BATCH)(table, idx)
    xf = x.reshape(BATCH, SEQ // 2, 2 * D_MODEL)
    out = _tc_add(xf, pos)
    return out.reshape(BATCH, SEQ, D_MODEL)


# speedup vs baseline: 5.4282x; 3.6010x over previous
import functools

import jax
import jax.numpy as jnp
from jax import lax
from jax.experimental import pallas as pl
from jax.experimental.pallas import tpu as pltpu
from jax.experimental.pallas import tpu_sc as plsc

D_MODEL = 64
MAX_LEN = 200
BATCH = 4096
SEQ = 200


@functools.lru_cache(maxsize=None)
def _make_sc_gather(V, D, B):
    NC, NS = 2, 16
    NW = NC * NS
    assert B % (8 * NW) == 0
    b_per_w = B // NW
    mesh = plsc.VectorSubcoreMesh(
        core_axis_name="c", subcore_axis_name="s", num_cores=NC, num_subcores=NS
    )

    @functools.partial(
        pl.kernel,
        mesh=mesh,
        out_type=jax.ShapeDtypeStruct((B, D), jnp.float32),
        scratch_types=[
            pltpu.VMEM((b_per_w,), jnp.int32),
            pltpu.VMEM((b_per_w, D), jnp.float32),
            pltpu.SemaphoreType.DMA,
        ],
    )
    def gather(table_hbm, idx_hbm, out_hbm, idx_v, rows_v, sem):
        wid = lax.axis_index("s") * NC + lax.axis_index("c")
        base = wid * b_per_w
        pltpu.sync_copy(idx_hbm.at[pl.ds(base, b_per_w)], idx_v)
        pltpu.async_copy(table_hbm.at[idx_v], rows_v, sem).wait()
        pltpu.sync_copy(rows_v, out_hbm.at[pl.ds(base, b_per_w)])

    return gather


def _add_body(x_ref, pos_ref, o_ref):
    o_ref[...] = x_ref[...] + pos_ref[...][None, :, :]


def _tc_add(xt, pos_t, seq_per_block=8):
    S, D, B = xt.shape
    grid = (S // seq_per_block,)
    return pl.pallas_call(
        _add_body,
        grid=grid,
        in_specs=[
            pl.BlockSpec((seq_per_block, D, B), lambda i: (i, 0, 0)),
            pl.BlockSpec((D, B), lambda i: (0, 0)),
        ],
        out_specs=pl.BlockSpec((seq_per_block, D, B), lambda i: (i, 0, 0)),
        out_shape=jax.ShapeDtypeStruct((S, D, B), xt.dtype),
    )(xt, pos_t)


@jax.jit
def kernel(x, pe, step_indices):
    idx = jnp.clip(step_indices.reshape(-1).astype(jnp.int32), 0, MAX_LEN - 1)
    table = jnp.pad(pe[0], ((0, 0), (0, 128 - D_MODEL)))
    pos = _make_sc_gather(MAX_LEN, 128, BATCH)(table, idx)
    pos_t = pos[:, :D_MODEL].T
    xt = x.transpose(1, 2, 0)
    out_t = _tc_add(xt, pos_t)
    return out_t.transpose(2, 0, 1)
